# baseline (speedup 1.0000x reference)
import functools

import jax
import jax.numpy as jnp
from jax import lax
from jax.experimental import pallas as pl
from jax.experimental.pallas import tpu as pltpu
from jax.experimental.pallas import tpu_sc as plsc

_TOPK = 8
_E = 64
_ROWS = 16384
_D = 2048
_RDIM = 16

_NCHUNK = 1
_CROWS = _ROWS // _NCHUNK
_NW = 32
_RPW = _CROWS // _NW
_BLK = _RPW
_GRP = 2
_LANES = 16


def _tc_scores_body(x_ref, w_ref, k_ref, s_ref, st_ref):
    xb = x_ref[...]
    w = w_ref[...]
    q = lax.dot_general(xb, w, (((1,), (1,)), ((), ())),
                        preferred_element_type=jnp.float32)
    qn = q * lax.rsqrt(jnp.maximum(jnp.sum(q * q, axis=-1, keepdims=True),
                                   1e-24))
    keys = k_ref[...]
    kn = keys * lax.rsqrt(
        jnp.maximum(jnp.sum(keys * keys, axis=-1, keepdims=True), 1e-24))
    s_ref[...] = lax.dot_general(qn, kn, (((1,), (1,)), ((), ())),
                                 preferred_element_type=jnp.float32)
    st_ref[...] = lax.dot_general(kn, qn, (((1,), (1,)), ((), ())),
                                  preferred_element_type=jnp.float32)[None]


def _tc_scores(x, W_query, keys):
    return pl.pallas_call(
        _tc_scores_body,
        grid=(_CROWS // _BLK,),
        in_specs=[
            pl.BlockSpec((_BLK, _D), lambda i: (i, 0)),
            pl.BlockSpec((_RDIM, _D), lambda i: (0, 0)),
            pl.BlockSpec((_E, _RDIM), lambda i: (0, 0)),
        ],
        out_specs=[
            pl.BlockSpec((_BLK, _E), lambda i: (i, 0)),
            pl.BlockSpec((1, _E, _BLK), lambda i: (i, 0, 0)),
        ],
        out_shape=[
            jax.ShapeDtypeStruct((_CROWS, _E), jnp.float32),
            jax.ShapeDtypeStruct((_NW, _E, _BLK), jnp.float32),
        ],
        compiler_params=pltpu.CompilerParams(
            dimension_semantics=("arbitrary",)),
    )(x, W_query, keys)


def _sc_topk_body(st_hbm, idx_hbm, p_hbm, sbuf, ibuf, pbuf):
    cid = lax.axis_index("c")
    sid = lax.axis_index("s")
    wid = sid * 2 + cid
    pltpu.sync_copy(st_hbm.at[wid], sbuf)

    lane = lax.iota(jnp.int32, _LANES)

    def group_step(i, carry):
        for g in range(_GRP):
            roff = i * (_LANES * _GRP) + g * _LANES
            row = roff + lane
            orow = row * _TOPK
            vals = [jnp.full((_LANES,), -jnp.inf, jnp.float32)
                    for _ in range(_TOPK)]
            for col in range(_E):
                v = sbuf[col, pl.ds(roff, _LANES)]
                for t in range(_TOPK):
                    new_val = jnp.maximum(v, vals[t])
                    v = jnp.minimum(v, vals[t])
                    vals[t] = new_val
            thresh = vals[_TOPK - 1]
            cnt = jnp.zeros((_LANES,), jnp.int32)
            for col in range(_E):
                v = sbuf[col, pl.ds(roff, _LANES)]
                m = (v >= thresh) & (cnt < _TOPK)
                off = orow + cnt
                plsc.store_scatter(ibuf, [off],
                                   jnp.full((_LANES,), col, jnp.int32),
                                   mask=m)
                plsc.store_scatter(pbuf, [off], v, mask=m)
                cnt = cnt + jnp.where(m, 1, 0)
            cv = [plsc.load_gather(pbuf, [orow + t]) for t in range(_TOPK)]
            ci = [plsc.load_gather(ibuf, [orow + t]) for t in range(_TOPK)]

            def ce(a, b):
                va, ia = cv[a], ci[a]
                vb, ib = cv[b], ci[b]
                m = (va > vb) | ((va == vb) & (ia < ib))
                cv[a] = jnp.where(m, va, vb)
                cv[b] = jnp.where(m, vb, va)
                ci[a] = jnp.where(m, ia, ib)
                ci[b] = jnp.where(m, ib, ia)

            for a, b in ((0, 1), (2, 3), (4, 5), (6, 7),
                         (0, 2), (1, 3), (4, 6), (5, 7),
                         (1, 2), (5, 6),
                         (0, 4), (1, 5), (2, 6), (3, 7),
                         (2, 4), (3, 5),
                         (1, 2), (3, 4), (5, 6)):
                ce(a, b)

            es = [jnp.exp(vt - cv[0]) for vt in cv]
            tot = es[0]
            for t in range(1, _TOPK):
                tot = tot + es[t]
            rcp = 1.0 / tot
            for t in range(_TOPK):
                plsc.store_scatter(pbuf, [orow + t], es[t] * rcp)
                plsc.store_scatter(ibuf, [orow + t], ci[t])
        return carry

    lax.fori_loop(0, _RPW // (_LANES * _GRP), group_step, 0)
    obase = wid * (_RPW * _TOPK)
    pltpu.sync_copy(ibuf, idx_hbm.at[pl.ds(obase, _RPW * _TOPK)])
    pltpu.sync_copy(pbuf, p_hbm.at[pl.ds(obase, _RPW * _TOPK)])


def _sc_topk(scores_t):
    f = functools.partial(
        pl.kernel,
        mesh=plsc.VectorSubcoreMesh(core_axis_name="c", subcore_axis_name="s"),
        compiler_params=pltpu.CompilerParams(needs_layout_passes=False),
        out_type=[
            jax.ShapeDtypeStruct((_CROWS * _TOPK,), jnp.int32),
            jax.ShapeDtypeStruct((_CROWS * _TOPK,), jnp.float32),
        ],
        scratch_types=[
            pltpu.VMEM((_E, _RPW), jnp.float32),
            pltpu.VMEM((_RPW * _TOPK,), jnp.int32),
            pltpu.VMEM((_RPW * _TOPK,), jnp.float32),
        ],
    )(_sc_topk_body)
    idx, p = f(scores_t)
    return (idx.reshape(_CROWS, _TOPK), p.reshape(_CROWS, _TOPK))


@jax.jit
def kernel(x, W_query, keys):
    scores_chunks, idx_chunks, p_chunks = [], [], []
    for c in range(_NCHUNK):
        xc = lax.slice_in_dim(x, c * _CROWS, (c + 1) * _CROWS, axis=0)
        scores_c, scores_t_c = _tc_scores(xc, W_query, keys)
        idx_c, p_c = _sc_topk(scores_t_c)
        scores_chunks.append(scores_c)
        idx_chunks.append(idx_c)
        p_chunks.append(p_c)
    scores = jnp.concatenate(scores_chunks, axis=0)
    idx = jnp.concatenate(idx_chunks, axis=0)
    probs = jnp.concatenate(p_chunks, axis=0)
    return (idx, probs, scores)


# speedup vs baseline: 1.0652x; 1.0652x over previous
import functools

import jax
import jax.numpy as jnp
from jax import lax
from jax.experimental import pallas as pl
from jax.experimental.pallas import tpu as pltpu
from jax.experimental.pallas import tpu_sc as plsc

_TOPK = 8
_E = 64
_ROWS = 16384
_D = 2048
_RDIM = 16

_NCHUNK = 1
_CROWS = _ROWS // _NCHUNK
_NW = 32
_RPW = _CROWS // _NW
_BLK = _RPW
_GRP = 2
_LANES = 16


def _tc_scores_body(x_ref, w_ref, k_ref, s_ref, st_ref):
    xb = x_ref[...]
    w = w_ref[...]
    q = lax.dot_general(xb, w, (((1,), (1,)), ((), ())),
                        preferred_element_type=jnp.float32)
    qn = q * lax.rsqrt(jnp.maximum(jnp.sum(q * q, axis=-1, keepdims=True),
                                   1e-24))
    keys = k_ref[...]
    kn = keys * lax.rsqrt(
        jnp.maximum(jnp.sum(keys * keys, axis=-1, keepdims=True), 1e-24))
    s_ref[...] = lax.dot_general(qn, kn, (((1,), (1,)), ((), ())),
                                 preferred_element_type=jnp.float32)
    st_ref[...] = lax.dot_general(kn, qn, (((1,), (1,)), ((), ())),
                                  preferred_element_type=jnp.float32)[None]


def _tc_scores(x, W_query, keys):
    return pl.pallas_call(
        _tc_scores_body,
        grid=(_CROWS // _BLK,),
        in_specs=[
            pl.BlockSpec((_BLK, _D), lambda i: (i, 0)),
            pl.BlockSpec((_RDIM, _D), lambda i: (0, 0)),
            pl.BlockSpec((_E, _RDIM), lambda i: (0, 0)),
        ],
        out_specs=[
            pl.BlockSpec((_BLK, _E), lambda i: (i, 0)),
            pl.BlockSpec((1, _E, _BLK), lambda i: (i, 0, 0)),
        ],
        out_shape=[
            jax.ShapeDtypeStruct((_CROWS, _E), jnp.float32),
            jax.ShapeDtypeStruct((_NW, _E, _BLK), jnp.float32),
        ],
        compiler_params=pltpu.CompilerParams(
            dimension_semantics=("arbitrary",)),
    )(x, W_query, keys)


def _sc_topk_body(st_hbm, idx_hbm, p_hbm, sbuf, ibuf, pbuf):
    cid = lax.axis_index("c")
    sid = lax.axis_index("s")
    wid = sid * 2 + cid
    pltpu.sync_copy(st_hbm.at[wid], sbuf)

    lane = lax.iota(jnp.int32, _LANES)

    def group_step(i, carry):
        for g in range(_GRP):
            roff = i * (_LANES * _GRP) + g * _LANES
            row = roff + lane
            orow = row * _TOPK
            vals = [jnp.full((_LANES,), -jnp.inf, jnp.float32)
                    for _ in range(_TOPK)]
            idxs = [jnp.zeros((_LANES,), jnp.int32) for _ in range(_TOPK)]
            for col in range(_E):
                v = sbuf[col, pl.ds(roff, _LANES)]
                vi = jnp.full((_LANES,), col, jnp.int32)
                for t in range(_TOPK):
                    m = v > vals[t]
                    new_val = jnp.maximum(v, vals[t])
                    v = jnp.minimum(v, vals[t])
                    vals[t] = new_val
                    new_idx = jnp.where(m, vi, idxs[t])
                    vi = jnp.where(m, idxs[t], vi)
                    idxs[t] = new_idx
            es = [jnp.exp(vt - vals[0]) for vt in vals]
            tot = es[0]
            for t in range(1, _TOPK):
                tot = tot + es[t]
            rcp = 1.0 / tot
            for t in range(_TOPK):
                plsc.store_scatter(pbuf, [orow + t], es[t] * rcp)
                plsc.store_scatter(ibuf, [orow + t], idxs[t])
        return carry

    lax.fori_loop(0, _RPW // (_LANES * _GRP), group_step, 0)
    obase = wid * (_RPW * _TOPK)
    pltpu.sync_copy(ibuf, idx_hbm.at[pl.ds(obase, _RPW * _TOPK)])
    pltpu.sync_copy(pbuf, p_hbm.at[pl.ds(obase, _RPW * _TOPK)])


def _sc_topk(scores_t):
    f = functools.partial(
        pl.kernel,
        mesh=plsc.VectorSubcoreMesh(core_axis_name="c", subcore_axis_name="s"),
        compiler_params=pltpu.CompilerParams(needs_layout_passes=False),
        out_type=[
            jax.ShapeDtypeStruct((_CROWS * _TOPK,), jnp.int32),
            jax.ShapeDtypeStruct((_CROWS * _TOPK,), jnp.float32),
        ],
        scratch_types=[
            pltpu.VMEM((_E, _RPW), jnp.float32),
            pltpu.VMEM((_RPW * _TOPK,), jnp.int32),
            pltpu.VMEM((_RPW * _TOPK,), jnp.float32),
        ],
    )(_sc_topk_body)
    idx, p = f(scores_t)
    return (idx.reshape(_CROWS, _TOPK), p.reshape(_CROWS, _TOPK))


@jax.jit
def kernel(x, W_query, keys):
    scores_chunks, idx_chunks, p_chunks = [], [], []
    for c in range(_NCHUNK):
        xc = lax.slice_in_dim(x, c * _CROWS, (c + 1) * _CROWS, axis=0)
        scores_c, scores_t_c = _tc_scores(xc, W_query, keys)
        idx_c, p_c = _sc_topk(scores_t_c)
        scores_chunks.append(scores_c)
        idx_chunks.append(idx_c)
        p_chunks.append(p_c)
    scores = jnp.concatenate(scores_chunks, axis=0)
    idx = jnp.concatenate(idx_chunks, axis=0)
    probs = jnp.concatenate(p_chunks, axis=0)
    return (idx, probs, scores)


# speedup vs baseline: 1.2748x; 1.1968x over previous
import functools

import jax
import jax.numpy as jnp
from jax import lax
from jax.experimental import pallas as pl
from jax.experimental.pallas import tpu as pltpu
from jax.experimental.pallas import tpu_sc as plsc

_TOPK = 8
_E = 64
_ROWS = 16384
_D = 2048
_RDIM = 16

_NCHUNK = 1
_CROWS = _ROWS // _NCHUNK
_NW = 32
_RPW = _CROWS // _NW
_BLK = _RPW
_GRP = 1
_LANES = 16


def _tc_scores_body(x_ref, w_ref, k_ref, s_ref, st_ref):
    xb = x_ref[...]
    w = w_ref[...]
    q = lax.dot_general(xb, w, (((1,), (1,)), ((), ())),
                        preferred_element_type=jnp.float32)
    qn = q * lax.rsqrt(jnp.maximum(jnp.sum(q * q, axis=-1, keepdims=True),
                                   1e-24))
    keys = k_ref[...]
    kn = keys * lax.rsqrt(
        jnp.maximum(jnp.sum(keys * keys, axis=-1, keepdims=True), 1e-24))
    s_ref[...] = lax.dot_general(qn, kn, (((1,), (1,)), ((), ())),
                                 preferred_element_type=jnp.float32)
    st_ref[...] = lax.dot_general(kn, qn, (((1,), (1,)), ((), ())),
                                  preferred_element_type=jnp.float32)[None]


def _tc_scores(x, W_query, keys):
    return pl.pallas_call(
        _tc_scores_body,
        grid=(_CROWS // _BLK,),
        in_specs=[
            pl.BlockSpec((_BLK, _D), lambda i: (i, 0)),
            pl.BlockSpec((_RDIM, _D), lambda i: (0, 0)),
            pl.BlockSpec((_E, _RDIM), lambda i: (0, 0)),
        ],
        out_specs=[
            pl.BlockSpec((_BLK, _E), lambda i: (i, 0)),
            pl.BlockSpec((1, _E, _BLK), lambda i: (i, 0, 0)),
        ],
        out_shape=[
            jax.ShapeDtypeStruct((_CROWS, _E), jnp.float32),
            jax.ShapeDtypeStruct((_NW, _E, _BLK), jnp.float32),
        ],
        compiler_params=pltpu.CompilerParams(
            dimension_semantics=("arbitrary",)),
    )(x, W_query, keys)


def _sc_topk_body(st_hbm, idx_hbm, p_hbm, sbuf, ibuf, pbuf):
    cid = lax.axis_index("c")
    sid = lax.axis_index("s")
    wid = sid * 2 + cid
    pltpu.sync_copy(st_hbm.at[wid], sbuf)

    lane = lax.iota(jnp.int32, _LANES)

    def group_step(i, carry):
        for g in range(_GRP):
            roff = i * (_LANES * _GRP) + g * _LANES
            row = roff + lane
            orow = row * _TOPK
            vals = [jnp.full((_LANES,), -jnp.inf, jnp.float32)
                    for _ in range(_TOPK)]
            idxs = [jnp.zeros((_LANES,), jnp.int32) for _ in range(_TOPK)]
            for col in range(_E):
                v = sbuf[col, pl.ds(roff, _LANES)]
                vi = jnp.full((_LANES,), col, jnp.int32)
                for t in range(_TOPK):
                    m = v > vals[t]
                    new_val = jnp.maximum(v, vals[t])
                    v = jnp.minimum(v, vals[t])
                    vals[t] = new_val
                    new_idx = jnp.where(m, vi, idxs[t])
                    vi = jnp.where(m, idxs[t], vi)
                    idxs[t] = new_idx
            es = [jnp.exp(vt - vals[0]) for vt in vals]
            tot = es[0]
            for t in range(1, _TOPK):
                tot = tot + es[t]
            rcp = 1.0 / tot
            for t in range(_TOPK):
                plsc.store_scatter(pbuf, [orow + t], es[t] * rcp)
                plsc.store_scatter(ibuf, [orow + t], idxs[t])
        return carry

    lax.fori_loop(0, _RPW // (_LANES * _GRP), group_step, 0)
    obase = wid * (_RPW * _TOPK)
    pltpu.sync_copy(ibuf, idx_hbm.at[pl.ds(obase, _RPW * _TOPK)])
    pltpu.sync_copy(pbuf, p_hbm.at[pl.ds(obase, _RPW * _TOPK)])


def _sc_topk(scores_t):
    f = functools.partial(
        pl.kernel,
        mesh=plsc.VectorSubcoreMesh(core_axis_name="c", subcore_axis_name="s"),
        compiler_params=pltpu.CompilerParams(needs_layout_passes=False),
        out_type=[
            jax.ShapeDtypeStruct((_CROWS * _TOPK,), jnp.int32),
            jax.ShapeDtypeStruct((_CROWS * _TOPK,), jnp.float32),
        ],
        scratch_types=[
            pltpu.VMEM((_E, _RPW), jnp.float32),
            pltpu.VMEM((_RPW * _TOPK,), jnp.int32),
            pltpu.VMEM((_RPW * _TOPK,), jnp.float32),
        ],
    )(_sc_topk_body)
    idx, p = f(scores_t)
    return (idx.reshape(_CROWS, _TOPK), p.reshape(_CROWS, _TOPK))


@jax.jit
def kernel(x, W_query, keys):
    scores_chunks, idx_chunks, p_chunks = [], [], []
    for c in range(_NCHUNK):
        xc = lax.slice_in_dim(x, c * _CROWS, (c + 1) * _CROWS, axis=0)
        scores_c, scores_t_c = _tc_scores(xc, W_query, keys)
        idx_c, p_c = _sc_topk(scores_t_c)
        scores_chunks.append(scores_c)
        idx_chunks.append(idx_c)
        p_chunks.append(p_c)
    scores = jnp.concatenate(scores_chunks, axis=0)
    idx = jnp.concatenate(idx_chunks, axis=0)
    probs = jnp.concatenate(p_chunks, axis=0)
    return (idx, probs, scores)
